# baseline (speedup 1.0000x reference)
import functools

import jax
import jax.numpy as jnp
from jax import lax
from jax.experimental import pallas as pl
from jax.experimental.pallas import tpu as pltpu
from jax.experimental.pallas import tpu_sc as plsc

DEPTH = 1000
BATCH = 1024
GROUP = 26
NC, NS, LANES = 2, 16, 16
NW = NC * NS
DCH = 40
NDC = DEPTH // DCH
NSLAB = GROUP * NDC
JMAX = (NSLAB + NW - 1) // NW

_mesh = plsc.VectorSubcoreMesh(core_axis_name="c", subcore_axis_name="s")


@functools.partial(
    pl.kernel,
    out_type=jax.ShapeDtypeStruct((GROUP, DEPTH, BATCH), jnp.float32),
    mesh=_mesh,
    compiler_params=pltpu.CompilerParams(needs_layout_passes=False),
    scratch_types=[
        pltpu.VMEM((1, BATCH), jnp.int32),
        pltpu.VMEM((1, BATCH), jnp.int32),
        pltpu.VMEM((1, DCH, BATCH), jnp.float32),
        pltpu.VMEM((1, DCH, BATCH), jnp.float32),
        pltpu.SemaphoreType.DMA,
        pltpu.SemaphoreType.DMA,
    ],
)
def _sc_onehot_t(idxt_hbm, zeros_hbm, out_hbm, col_v, colp_v, buf0, buf1,
                 sem0, sem1):
    wid = lax.axis_index("s") * NC + lax.axis_index("c")
    pltpu.sync_copy(zeros_hbm, buf0)
    pltpu.sync_copy(zeros_hbm, buf1)

    bufs = (buf0, buf1)
    sems = (sem0, sem1)
    ones = jnp.full((LANES,), 1.0, jnp.float32)
    zsf = jnp.zeros((LANES,), jnp.float32)
    z16 = jnp.zeros((LANES,), jnp.int32)
    lane = lax.iota(jnp.int32, LANES)

    def params(j):
        s = wid + NW * j
        g = s // NDC
        d0 = (s - g * NDC) * DCH
        return s, g, d0

    def scatter_slab(buf, col, d0, val):
        def body(k, carry):
            v16 = col[0, pl.ds(LANES * k, LANES)]
            b16 = lane + LANES * k
            m = (v16 >= d0) & (v16 < d0 + DCH)
            plsc.store_scatter(buf, [z16, v16 - d0, b16], val, mask=m)
            return carry
        lax.fori_loop(0, BATCH // LANES, body, 0)

    def dma(buf, g, d0, sem):
        return pltpu.make_async_copy(
            buf, out_hbm.at[pl.ds(g, 1), pl.ds(d0, DCH)], sem)

    for j in range(JMAX):
        b = j % 2
        buf = bufs[b]
        if j >= 2:
            sp, gp, d0p = params(j - 2)

            @pl.when(sp < NSLAB)
            def _():
                dma(buf, 0, 0, sems[b]).wait()
                pltpu.sync_copy(idxt_hbm.at[pl.ds(gp, 1)], colp_v)
                scatter_slab(buf, colp_v, d0p, zsf)

        s, g, d0 = params(j)

        @pl.when(s < NSLAB)
        def _():
            pltpu.sync_copy(idxt_hbm.at[pl.ds(g, 1)], col_v)
            scatter_slab(buf, col_v, d0, ones)
            dma(buf, g, d0, sems[b]).start()

    for j in (JMAX - 2, JMAX - 1):
        s, g, d0 = params(j)

        @pl.when(s < NSLAB)
        def _():
            dma(bufs[j % 2], 0, 0, sems[j % 2]).wait()


def kernel(inputs):
    idxt = inputs.T
    zeros = jnp.zeros((1, DCH, BATCH), jnp.float32)
    out_t = _sc_onehot_t(idxt, zeros)
    return out_t.transpose(2, 0, 1)


# speedup vs baseline: 3.2685x; 3.2685x over previous
import jax
import jax.numpy as jnp
from jax import lax
from jax.experimental import pallas as pl

DEPTH = 1000
BATCH = 1024
GROUP = 26
BLOCK_D = 40


def _onehot_body(idxt_ref, out_ref):
    idxt = idxt_ref[...]
    d0 = pl.program_id(0) * BLOCK_D
    dio = d0 + lax.broadcasted_iota(jnp.int32, (GROUP, BLOCK_D, BATCH), 1)
    out_ref[...] = (idxt[:, None, :] == dio).astype(jnp.float32)


def kernel(inputs):
    idxt = inputs.T
    out_t = pl.pallas_call(
        _onehot_body,
        grid=(DEPTH // BLOCK_D,),
        in_specs=[pl.BlockSpec((GROUP, BATCH), lambda i: (0, 0))],
        out_specs=pl.BlockSpec((GROUP, BLOCK_D, BATCH), lambda i: (0, i, 0)),
        out_shape=jax.ShapeDtypeStruct((GROUP, DEPTH, BATCH), jnp.float32),
    )(idxt)
    return out_t.transpose(2, 0, 1)
